# Initial kernel scaffold; baseline (speedup 1.0000x reference)
import jax
import jax.numpy as jnp
from jax.experimental import pallas as pl


def kernel(sents_batch, table, W, b):
    raise NotImplementedError("write your pallas kernel here")


# speedup vs baseline: 8.3206x; 8.3206x over previous
import functools

import jax
import jax.numpy as jnp
from jax import lax
from jax.experimental import pallas as pl
from jax.experimental.pallas import tpu as pltpu
from jax.experimental.pallas import tpu_sc as plsc

VOCAB = 100000
EMB = 64
CLASSES = 128
BATCH = 1024
TOKENS = 1000
NCORES = 2
NSUB = 16
NW = NCORES * NSUB
EPW = BATCH // NW
NCHUNK = 8
CH = 128
REAL = 125


def _sc_bag_sums(table, idx4):
    mesh = plsc.VectorSubcoreMesh(core_axis_name="c", subcore_axis_name="s")

    @functools.partial(
        pl.kernel,
        mesh=mesh,
        compiler_params=pltpu.CompilerParams(use_tc_tiling_on_sc=False),
        out_type=jax.ShapeDtypeStruct((BATCH, EMB), jnp.float32),
        scratch_types=[
            pltpu.VMEM((EPW, NCHUNK, CH), jnp.int32),
            pltpu.VMEM((CH, EMB), jnp.float32),
            pltpu.VMEM((EPW, EMB), jnp.float32),
            pltpu.SemaphoreType.DMA,
        ],
    )
    def k(table_hbm, idx_hbm, out_hbm, idx_v, rows_v, out_v, sem):
        wid = lax.axis_index("s") * NCORES + lax.axis_index("c")
        pltpu.sync_copy(idx_hbm.at[wid], idx_v)

        def bag(e, _):
            def chunk(c, accs):
                pltpu.async_copy(table_hbm.at[idx_v.at[e, c]], rows_v, sem).wait()

                def row(r, accs):
                    return tuple(
                        accs[i] + rows_v[r, pl.ds(i * 16, 16)] for i in range(4)
                    )

                return lax.fori_loop(0, REAL, row, accs)

            accs = tuple(jnp.zeros((16,), jnp.float32) for _ in range(4))
            accs = lax.fori_loop(0, NCHUNK, chunk, accs)
            for i in range(4):
                out_v[e, pl.ds(i * 16, 16)] = accs[i]
            return 0

        lax.fori_loop(0, EPW, bag, 0)
        pltpu.sync_copy(out_v, out_hbm.at[pl.ds(wid * EPW, EPW)])

    return k(table, idx4)


def _tc_linear(sums, W, b2d):
    def body(x_ref, w_ref, b_ref, o_ref):
        acc = lax.dot_general(
            x_ref[...], w_ref[...],
            (((1,), (1,)), ((), ())),
            preferred_element_type=jnp.float32,
        )
        o_ref[...] = acc * (1.0 / TOKENS) + b_ref[...]

    return pl.pallas_call(
        body,
        out_shape=jax.ShapeDtypeStruct((BATCH, CLASSES), jnp.float32),
    )(sums, W, b2d)


def kernel(sents_batch, table, W, b):
    idx = sents_batch.reshape(BATCH, NCHUNK, REAL).astype(jnp.int32)
    idx = jnp.pad(idx, ((0, 0), (0, 0), (0, CH - REAL)))
    idx4 = idx.reshape(NW, EPW, NCHUNK, CH)
    sums = _sc_bag_sums(table, idx4)
    return _tc_linear(sums, W, b.reshape(1, CLASSES))
